# Initial kernel scaffold; baseline (speedup 1.0000x reference)
import functools

import jax
import jax.numpy as jnp
from jax import lax
from jax.experimental import pallas as pl
from jax.experimental.pallas import tpu as pltpu
from jax.experimental.pallas import tpu_sc as plsc

_N, _E, _D = 10000, 320000, 128
_NC, _NS = 2, 16
_NW = _NC * _NS
_EPW = _E // _NW
_CH = 80
_NCHUNK = _EPW // _CH
_RPT = _N // _NS
_BE = 4000
_GE = _E // _BE
_BN = 2000

_mesh = plsc.VectorSubcoreMesh(
    core_axis_name="c", subcore_axis_name="s", num_cores=_NC, num_subcores=_NS)


def _node_mm_body(h_ref, w_ref, qh_ref, rh_ref, vh_ref, uh_ref):
    p = jnp.dot(h_ref[...], w_ref[...], preferred_element_type=jnp.float32)
    qh_ref[...] = p[:, 0 * _D:1 * _D]
    rh_ref[...] = p[:, 1 * _D:2 * _D]
    vh_ref[...] = p[:, 2 * _D:3 * _D]
    uh_ref[...] = p[:, 3 * _D:4 * _D]


_node_mm = pl.pallas_call(
    _node_mm_body,
    grid=(_N // _BN,),
    in_specs=[
        pl.BlockSpec((_BN, _D), lambda i: (i, 0)),
        pl.BlockSpec((_D, 4 * _D), lambda i: (0, 0)),
    ],
    out_specs=[pl.BlockSpec((_BN, _D), lambda i: (i, 0))] * 4,
    out_shape=[jax.ShapeDtypeStruct((_N, _D), jnp.float32)] * 4,
)


def _edge1_body(pT_ref, e_ref, gq_ref, gr_ref, gv_ref,
                ehat_ref, msg_ref, stats_ref):
    e_hat = (jnp.dot(e_ref[...], pT_ref[...], preferred_element_type=jnp.float32)
             + gq_ref[...] + gr_ref[...])
    ehat_ref[...] = e_hat
    msg_ref[...] = jax.nn.sigmoid(e_hat) * gv_ref[...]
    s1 = jnp.sum(e_hat, axis=0, keepdims=True)
    s2 = jnp.sum(e_hat * e_hat, axis=0, keepdims=True)
    blk = jnp.concatenate([s1, s2, jnp.zeros((6, _D), jnp.float32)], axis=0)

    @pl.when(pl.program_id(0) == 0)
    def _():
        stats_ref[...] = blk

    @pl.when(pl.program_id(0) != 0)
    def _():
        stats_ref[...] += blk


_edge_pass1 = pl.pallas_call(
    _edge1_body,
    grid=(_GE,),
    in_specs=[
        pl.BlockSpec((_D, _D), lambda i: (0, 0)),
        pl.BlockSpec((_BE, _D), lambda i: (i, 0)),
        pl.BlockSpec((_BE, _D), lambda i: (i, 0)),
        pl.BlockSpec((_BE, _D), lambda i: (i, 0)),
        pl.BlockSpec((_BE, _D), lambda i: (i, 0)),
    ],
    out_specs=[
        pl.BlockSpec((_BE, _D), lambda i: (i, 0)),
        pl.BlockSpec((_BE, _D), lambda i: (i, 0)),
        pl.BlockSpec((8, _D), lambda i: (0, 0)),
    ],
    out_shape=[
        jax.ShapeDtypeStruct((_E, _D), jnp.float32),
        jax.ShapeDtypeStruct((_E, _D), jnp.float32),
        jax.ShapeDtypeStruct((8, _D), jnp.float32),
    ],
)


def _edge2_body(stats_ref, m1T_ref, m2T_ref, b1_ref, b2_ref, g_ref, bta_ref,
                e_ref, ehat_ref, enew_ref):
    mean = stats_ref[0:1, :] * (1.0 / _E)
    var = stats_ref[1:2, :] * (1.0 / _E) - mean * mean
    rstd = lax.rsqrt(var + 1e-5)
    x = (ehat_ref[...] - mean) * rstd * g_ref[...] + bta_ref[...]
    t = jnp.maximum(
        jnp.dot(x, m1T_ref[...], preferred_element_type=jnp.float32)
        + b1_ref[...], 0.0)
    y = jnp.dot(t, m2T_ref[...], preferred_element_type=jnp.float32) + b2_ref[...]
    enew_ref[...] = e_ref[...] + y


_edge_pass2 = pl.pallas_call(
    _edge2_body,
    grid=(_GE,),
    in_specs=[
        pl.BlockSpec((8, _D), lambda i: (0, 0)),
        pl.BlockSpec((_D, _D), lambda i: (0, 0)),
        pl.BlockSpec((_D, _D), lambda i: (0, 0)),
        pl.BlockSpec((1, _D), lambda i: (0, 0)),
        pl.BlockSpec((1, _D), lambda i: (0, 0)),
        pl.BlockSpec((1, _D), lambda i: (0, 0)),
        pl.BlockSpec((1, _D), lambda i: (0, 0)),
        pl.BlockSpec((_BE, _D), lambda i: (i, 0)),
        pl.BlockSpec((_BE, _D), lambda i: (i, 0)),
    ],
    out_specs=pl.BlockSpec((_BE, _D), lambda i: (i, 0)),
    out_shape=jax.ShapeDtypeStruct((_E, _D), jnp.float32),
)


def _node_bn_body(h_ref, uh_ref, a0_ref, a1_ref, g_ref, bta_ref, hnew_ref):
    z = uh_ref[...] + a0_ref[...] + a1_ref[...]
    mu = jnp.mean(z, axis=0, keepdims=True)
    var = jnp.mean((z - mu) * (z - mu), axis=0, keepdims=True)
    zn = (z - mu) * lax.rsqrt(var + 1e-5) * g_ref[...] + bta_ref[...]
    hnew_ref[...] = h_ref[...] + zn


_node_pass = pl.pallas_call(
    _node_bn_body,
    out_shape=jax.ShapeDtypeStruct((_N, _D), jnp.float32),
)


@functools.partial(
    pl.kernel,
    mesh=_mesh,
    out_type=[jax.ShapeDtypeStruct((_E, _D), jnp.float32)] * 3,
    scratch_types=[
        pltpu.VMEM((_CH,), jnp.int32),
        pltpu.VMEM((_CH,), jnp.int32),
        pltpu.VMEM((_CH, _D), jnp.float32),
        pltpu.VMEM((_CH, _D), jnp.float32),
        pltpu.VMEM((_CH, _D), jnp.float32),
        pltpu.SemaphoreType.DMA,
    ],
)
def _gather(qh, rh, vh, src, dst, gq, gr, gv, src_v, dst_v, qb, rb, vb, sem):
    wid = lax.axis_index("s") * _NC + lax.axis_index("c")
    base0 = wid * _EPW

    def body(i, carry):
        base = base0 + i * _CH
        pltpu.sync_copy(src.at[pl.ds(base, _CH)], src_v)
        pltpu.sync_copy(dst.at[pl.ds(base, _CH)], dst_v)
        c1 = pltpu.async_copy(qh.at[src_v], qb, sem)
        c2 = pltpu.async_copy(rh.at[dst_v], rb, sem)
        c3 = pltpu.async_copy(vh.at[dst_v], vb, sem)
        c1.wait()
        c2.wait()
        c3.wait()
        pltpu.sync_copy(qb, gq.at[pl.ds(base, _CH)])
        pltpu.sync_copy(rb, gr.at[pl.ds(base, _CH)])
        pltpu.sync_copy(vb, gv.at[pl.ds(base, _CH)])
        return carry

    lax.fori_loop(0, _NCHUNK, body, 0)


@functools.partial(
    pl.kernel,
    mesh=_mesh,
    out_type=jax.ShapeDtypeStruct((_NC, _N, _D), jnp.float32),
    scratch_types=[
        pltpu.VMEM((_CH,), jnp.int32),
        pltpu.VMEM((_CH, _D), jnp.float32),
        pltpu.VMEM_SHARED((_N, _D), jnp.float32),
    ],
)
def _scatter(msg, src, zeros_hbm, out, src_v, mb, agg_sh):
    c = lax.axis_index("c")
    s = lax.axis_index("s")
    wid = s * _NC + c
    pltpu.sync_copy(zeros_hbm.at[pl.ds(s * _RPT, _RPT)],
                    agg_sh.at[pl.ds(s * _RPT, _RPT)])
    plsc.subcore_barrier()
    base0 = wid * _EPW

    def body(i, carry):
        base = base0 + i * _CH
        pltpu.sync_copy(src.at[pl.ds(base, _CH)], src_v)
        pltpu.sync_copy(msg.at[pl.ds(base, _CH)], mb)
        pltpu.sync_copy(mb, agg_sh.at[src_v], add=True)
        return carry

    lax.fori_loop(0, _NCHUNK, body, 0)
    plsc.subcore_barrier()
    pltpu.sync_copy(agg_sh.at[pl.ds(s * _RPT, _RPT)],
                    out.at[c, pl.ds(s * _RPT, _RPT)])


def kernel(h, e, edge_index, P_w, Q_w, R_w, U_w, V_w,
           m1_w, m1_b, m2_w, m2_b, e_gamma, e_beta, n_gamma, n_beta):
    src = edge_index[0]
    dst = edge_index[1]
    w4 = jnp.concatenate([Q_w.T, R_w.T, V_w.T, U_w.T], axis=1)
    qh, rh, vh, uh = _node_mm(h, w4)
    gq, gr, gv = _gather(qh, rh, vh, src, dst)
    ehat, msg, stats = _edge_pass1(P_w.T, e, gq, gr, gv)
    aggc = _scatter(msg, src, jnp.zeros((_N, _D), jnp.float32))
    e_new = _edge_pass2(stats, m1_w.T, m2_w.T,
                        m1_b.reshape(1, _D), m2_b.reshape(1, _D),
                        e_gamma.reshape(1, _D), e_beta.reshape(1, _D),
                        e, ehat)
    h_new = _node_pass(h, uh, aggc[0], aggc[1],
                       n_gamma.reshape(1, _D), n_beta.reshape(1, _D))
    return (h_new, e_new)


# speedup vs baseline: 3.0626x; 3.0626x over previous
import functools

import jax
import jax.numpy as jnp
from jax import lax
from jax.experimental import pallas as pl
from jax.experimental.pallas import tpu as pltpu
from jax.experimental.pallas import tpu_sc as plsc

_N, _E, _D = 10000, 320000, 128
_NC, _NS = 2, 16
_NW = _NC * _NS
_EPW = _E // _NW
_CH = 80
_NCHUNK = _EPW // _CH
_NP = 10240
_RPT = _NP // _NS
_BE = 4000
_GE = _E // _BE
_BN = 2000

@functools.lru_cache(maxsize=1)
def _sc_mesh():
    return plsc.VectorSubcoreMesh(
        core_axis_name="c", subcore_axis_name="s",
        num_cores=_NC, num_subcores=_NS)


def _node_mm_body(h_ref, w_ref, qh_ref, rh_ref, vh_ref, uh_ref):
    p = jnp.dot(h_ref[...], w_ref[...], preferred_element_type=jnp.float32)
    qh_ref[...] = p[:, 0 * _D:1 * _D]
    rh_ref[...] = p[:, 1 * _D:2 * _D]
    vh_ref[...] = p[:, 2 * _D:3 * _D]
    uh_ref[...] = p[:, 3 * _D:4 * _D]


_node_mm = pl.pallas_call(
    _node_mm_body,
    grid=(_N // _BN,),
    in_specs=[
        pl.BlockSpec((_BN, _D), lambda i: (i, 0)),
        pl.BlockSpec((_D, 4 * _D), lambda i: (0, 0)),
    ],
    out_specs=[pl.BlockSpec((_BN, _D), lambda i: (i, 0))] * 4,
    out_shape=[jax.ShapeDtypeStruct((_N, _D), jnp.float32)] * 4,
)


def _edge1_body(pT_ref, e_ref, gq_ref, gr_ref, gv_ref,
                ehat_ref, msg_ref, stats_ref):
    e_hat = (jnp.dot(e_ref[...], pT_ref[...], preferred_element_type=jnp.float32)
             + gq_ref[...] + gr_ref[...])
    ehat_ref[...] = e_hat
    msg_ref[...] = jax.nn.sigmoid(e_hat) * gv_ref[...]
    s1 = jnp.sum(e_hat, axis=0, keepdims=True)
    s2 = jnp.sum(e_hat * e_hat, axis=0, keepdims=True)
    blk = jnp.concatenate([s1, s2, jnp.zeros((6, _D), jnp.float32)], axis=0)

    @pl.when(pl.program_id(0) == 0)
    def _():
        stats_ref[...] = blk

    @pl.when(pl.program_id(0) != 0)
    def _():
        stats_ref[...] += blk


_edge_pass1 = pl.pallas_call(
    _edge1_body,
    grid=(_GE,),
    in_specs=[
        pl.BlockSpec((_D, _D), lambda i: (0, 0)),
        pl.BlockSpec((_BE, _D), lambda i: (i, 0)),
        pl.BlockSpec((_BE, _D), lambda i: (i, 0)),
        pl.BlockSpec((_BE, _D), lambda i: (i, 0)),
        pl.BlockSpec((_BE, _D), lambda i: (i, 0)),
    ],
    out_specs=[
        pl.BlockSpec((_BE, _D), lambda i: (i, 0)),
        pl.BlockSpec((_BE, _D), lambda i: (i, 0)),
        pl.BlockSpec((8, _D), lambda i: (0, 0)),
    ],
    out_shape=[
        jax.ShapeDtypeStruct((_E, _D), jnp.float32),
        jax.ShapeDtypeStruct((_E, _D), jnp.float32),
        jax.ShapeDtypeStruct((8, _D), jnp.float32),
    ],
)


def _edge2_body(stats_ref, m1T_ref, m2T_ref, b1_ref, b2_ref, g_ref, bta_ref,
                e_ref, ehat_ref, enew_ref):
    mean = stats_ref[0:1, :] * (1.0 / _E)
    var = stats_ref[1:2, :] * (1.0 / _E) - mean * mean
    rstd = lax.rsqrt(var + 1e-5)
    x = (ehat_ref[...] - mean) * rstd * g_ref[...] + bta_ref[...]
    t = jnp.maximum(
        jnp.dot(x, m1T_ref[...], preferred_element_type=jnp.float32)
        + b1_ref[...], 0.0)
    y = jnp.dot(t, m2T_ref[...], preferred_element_type=jnp.float32) + b2_ref[...]
    enew_ref[...] = e_ref[...] + y


_edge_pass2 = pl.pallas_call(
    _edge2_body,
    grid=(_GE,),
    in_specs=[
        pl.BlockSpec((8, _D), lambda i: (0, 0)),
        pl.BlockSpec((_D, _D), lambda i: (0, 0)),
        pl.BlockSpec((_D, _D), lambda i: (0, 0)),
        pl.BlockSpec((1, _D), lambda i: (0, 0)),
        pl.BlockSpec((1, _D), lambda i: (0, 0)),
        pl.BlockSpec((1, _D), lambda i: (0, 0)),
        pl.BlockSpec((1, _D), lambda i: (0, 0)),
        pl.BlockSpec((_BE, _D), lambda i: (i, 0)),
        pl.BlockSpec((_BE, _D), lambda i: (i, 0)),
    ],
    out_specs=pl.BlockSpec((_BE, _D), lambda i: (i, 0)),
    out_shape=jax.ShapeDtypeStruct((_E, _D), jnp.float32),
)


def _node_bn_body(h_ref, uh_ref, a0_ref, a1_ref, g_ref, bta_ref, hnew_ref):
    z = uh_ref[...] + a0_ref[...] + a1_ref[...]
    mu = jnp.mean(z, axis=0, keepdims=True)
    var = jnp.mean((z - mu) * (z - mu), axis=0, keepdims=True)
    zn = (z - mu) * lax.rsqrt(var + 1e-5) * g_ref[...] + bta_ref[...]
    hnew_ref[...] = h_ref[...] + zn


_node_pass = pl.pallas_call(
    _node_bn_body,
    out_shape=jax.ShapeDtypeStruct((_N, _D), jnp.float32),
)


@functools.lru_cache(maxsize=1)
def _make_gather():
    @functools.partial(
        pl.kernel,
        mesh=_sc_mesh(),
        out_type=[jax.ShapeDtypeStruct((_E, _D), jnp.float32)] * 3,
        scratch_types=[
            pltpu.VMEM((_CH,), jnp.int32),
            pltpu.VMEM((_CH,), jnp.int32),
            pltpu.VMEM((_CH, _D), jnp.float32),
            pltpu.VMEM((_CH, _D), jnp.float32),
            pltpu.VMEM((_CH, _D), jnp.float32),
            pltpu.SemaphoreType.DMA,
        ],
    )
    def _gather(qh, rh, vh, src, dst, gq, gr, gv,
                src_v, dst_v, qb, rb, vb, sem):
        wid = lax.axis_index("s") * _NC + lax.axis_index("c")
        base0 = wid * _EPW

        def body(i, carry):
            base = base0 + i * _CH
            pltpu.sync_copy(src.at[pl.ds(base, _CH)], src_v)
            pltpu.sync_copy(dst.at[pl.ds(base, _CH)], dst_v)
            c1 = pltpu.async_copy(qh.at[src_v], qb, sem)
            c2 = pltpu.async_copy(rh.at[dst_v], rb, sem)
            c3 = pltpu.async_copy(vh.at[dst_v], vb, sem)
            c1.wait()
            c2.wait()
            c3.wait()
            pltpu.sync_copy(qb, gq.at[pl.ds(base, _CH)])
            pltpu.sync_copy(rb, gr.at[pl.ds(base, _CH)])
            pltpu.sync_copy(vb, gv.at[pl.ds(base, _CH)])
            return carry

        lax.fori_loop(0, _NCHUNK, body, 0)

    return _gather


@functools.lru_cache(maxsize=1)
def _make_scatter():
    @functools.partial(
        pl.kernel,
        mesh=_sc_mesh(),
        out_type=jax.ShapeDtypeStruct((_NC, _NP, _D), jnp.float32),
        scratch_types=[
            pltpu.VMEM((_CH,), jnp.int32),
            pltpu.VMEM((_CH, _D), jnp.float32),
            pltpu.VMEM_SHARED((_NP, _D), jnp.float32),
        ],
    )
    def _scatter(msg, src, zeros_hbm, out, src_v, mb, agg_sh):
        c = lax.axis_index("c")
        s = lax.axis_index("s")
        wid = s * _NC + c
        pltpu.sync_copy(zeros_hbm.at[pl.ds(s * _RPT, _RPT)],
                        agg_sh.at[pl.ds(s * _RPT, _RPT)])
        plsc.subcore_barrier()
        base0 = wid * _EPW

        def body(i, carry):
            base = base0 + i * _CH
            pltpu.sync_copy(src.at[pl.ds(base, _CH)], src_v)
            pltpu.sync_copy(msg.at[pl.ds(base, _CH)], mb)
            pltpu.sync_copy(mb, agg_sh.at[src_v], add=True)
            return carry

        lax.fori_loop(0, _NCHUNK, body, 0)
        plsc.subcore_barrier()
        pltpu.sync_copy(agg_sh.at[pl.ds(s * _RPT, _RPT)],
                        out.at[c, pl.ds(s * _RPT, _RPT)])

    return _scatter


def kernel(h, e, edge_index, P_w, Q_w, R_w, U_w, V_w,
           m1_w, m1_b, m2_w, m2_b, e_gamma, e_beta, n_gamma, n_beta):
    src = edge_index[0]
    dst = edge_index[1]
    w4 = jnp.concatenate([Q_w.T, R_w.T, V_w.T, U_w.T], axis=1)
    qh, rh, vh, uh = _node_mm(h, w4)
    gq, gr, gv = _make_gather()(qh, rh, vh, src, dst)
    ehat, msg, stats = _edge_pass1(P_w.T, e, gq, gr, gv)
    aggc = _make_scatter()(msg, src, jnp.zeros((_NP, _D), jnp.float32))
    e_new = _edge_pass2(stats, m1_w.T, m2_w.T,
                        m1_b.reshape(1, _D), m2_b.reshape(1, _D),
                        e_gamma.reshape(1, _D), e_beta.reshape(1, _D),
                        e, ehat)
    h_new = _node_pass(h, uh, aggc[0, :_N], aggc[1, :_N],
                       n_gamma.reshape(1, _D), n_beta.reshape(1, _D))
    return (h_new, e_new)


# speedup vs baseline: 3.2014x; 1.0453x over previous
import functools

import jax
import jax.numpy as jnp
from jax import lax
from jax.experimental import pallas as pl
from jax.experimental.pallas import tpu as pltpu
from jax.experimental.pallas import tpu_sc as plsc

_N, _E, _D = 10000, 320000, 128
_NC, _NS = 2, 16
_NW = _NC * _NS
_EPW = _E // _NW
_CH = 80
_NCHUNK = _EPW // _CH
_NP = 10240
_RPT = _NP // _NS
_BE = 4000
_GE = _E // _BE
_BN = 2000

@functools.lru_cache(maxsize=1)
def _sc_mesh():
    return plsc.VectorSubcoreMesh(
        core_axis_name="c", subcore_axis_name="s",
        num_cores=_NC, num_subcores=_NS)


def _node_mm_body(h_ref, w_ref, qh_ref, rh_ref, vh_ref, uh_ref):
    p = jnp.dot(h_ref[...], w_ref[...], preferred_element_type=jnp.float32)
    qh_ref[...] = p[:, 0 * _D:1 * _D]
    rh_ref[...] = p[:, 1 * _D:2 * _D]
    vh_ref[...] = p[:, 2 * _D:3 * _D]
    uh_ref[...] = p[:, 3 * _D:4 * _D]


_node_mm = pl.pallas_call(
    _node_mm_body,
    grid=(_N // _BN,),
    in_specs=[
        pl.BlockSpec((_BN, _D), lambda i: (i, 0)),
        pl.BlockSpec((_D, 4 * _D), lambda i: (0, 0)),
    ],
    out_specs=[pl.BlockSpec((_BN, _D), lambda i: (i, 0))] * 4,
    out_shape=[jax.ShapeDtypeStruct((_N, _D), jnp.float32)] * 4,
)


def _edge1_body(pT_ref, e_ref, g1_ref, ehat_ref, stats_ref):
    e_hat = (jnp.dot(e_ref[...], pT_ref[...], preferred_element_type=jnp.float32)
             + g1_ref[...])
    ehat_ref[...] = e_hat
    s1 = jnp.sum(e_hat, axis=0, keepdims=True)
    s2 = jnp.sum(e_hat * e_hat, axis=0, keepdims=True)
    blk = jnp.concatenate([s1, s2, jnp.zeros((6, _D), jnp.float32)], axis=0)

    @pl.when(pl.program_id(0) == 0)
    def _():
        stats_ref[...] = blk

    @pl.when(pl.program_id(0) != 0)
    def _():
        stats_ref[...] += blk


_edge_pass1 = pl.pallas_call(
    _edge1_body,
    grid=(_GE,),
    in_specs=[
        pl.BlockSpec((_D, _D), lambda i: (0, 0)),
        pl.BlockSpec((_BE, _D), lambda i: (i, 0)),
        pl.BlockSpec((_BE, _D), lambda i: (i, 0)),
    ],
    out_specs=[
        pl.BlockSpec((_BE, _D), lambda i: (i, 0)),
        pl.BlockSpec((8, _D), lambda i: (0, 0)),
    ],
    out_shape=[
        jax.ShapeDtypeStruct((_E, _D), jnp.float32),
        jax.ShapeDtypeStruct((8, _D), jnp.float32),
    ],
)


def _edge2_body(stats_ref, m1T_ref, m2T_ref, b1_ref, b2_ref, g_ref, bta_ref,
                e_ref, ehat_ref, enew_ref):
    mean = stats_ref[0:1, :] * (1.0 / _E)
    var = stats_ref[1:2, :] * (1.0 / _E) - mean * mean
    rstd = lax.rsqrt(var + 1e-5)
    x = (ehat_ref[...] - mean) * rstd * g_ref[...] + bta_ref[...]
    t = jnp.maximum(
        jnp.dot(x, m1T_ref[...], preferred_element_type=jnp.float32)
        + b1_ref[...], 0.0)
    y = jnp.dot(t, m2T_ref[...], preferred_element_type=jnp.float32) + b2_ref[...]
    enew_ref[...] = e_ref[...] + y


_edge_pass2 = pl.pallas_call(
    _edge2_body,
    grid=(_GE,),
    in_specs=[
        pl.BlockSpec((8, _D), lambda i: (0, 0)),
        pl.BlockSpec((_D, _D), lambda i: (0, 0)),
        pl.BlockSpec((_D, _D), lambda i: (0, 0)),
        pl.BlockSpec((1, _D), lambda i: (0, 0)),
        pl.BlockSpec((1, _D), lambda i: (0, 0)),
        pl.BlockSpec((1, _D), lambda i: (0, 0)),
        pl.BlockSpec((1, _D), lambda i: (0, 0)),
        pl.BlockSpec((_BE, _D), lambda i: (i, 0)),
        pl.BlockSpec((_BE, _D), lambda i: (i, 0)),
    ],
    out_specs=pl.BlockSpec((_BE, _D), lambda i: (i, 0)),
    out_shape=jax.ShapeDtypeStruct((_E, _D), jnp.float32),
)


def _node_bn_body(h_ref, uh_ref, a0_ref, a1_ref, g_ref, bta_ref, hnew_ref):
    z = uh_ref[...] + a0_ref[...] + a1_ref[...]
    mu = jnp.mean(z, axis=0, keepdims=True)
    var = jnp.mean((z - mu) * (z - mu), axis=0, keepdims=True)
    zn = (z - mu) * lax.rsqrt(var + 1e-5) * g_ref[...] + bta_ref[...]
    hnew_ref[...] = h_ref[...] + zn


_node_pass = pl.pallas_call(
    _node_bn_body,
    out_shape=jax.ShapeDtypeStruct((_N, _D), jnp.float32),
)


@functools.lru_cache(maxsize=1)
def _make_gather():
    @functools.partial(
        pl.kernel,
        mesh=_sc_mesh(),
        out_type=jax.ShapeDtypeStruct((_E, _D), jnp.float32),
        scratch_types=[
            pltpu.VMEM((_CH,), jnp.int32),
            pltpu.VMEM((_CH,), jnp.int32),
            pltpu.VMEM((_CH, _D), jnp.float32),
            pltpu.VMEM((_CH, _D), jnp.float32),
            pltpu.SemaphoreType.DMA,
        ],
    )
    def _gather(qh, rh, src, dst, g1, src_v, dst_v, qb, rb, sem):
        wid = lax.axis_index("s") * _NC + lax.axis_index("c")
        base0 = wid * _EPW

        def body(i, carry):
            base = base0 + i * _CH
            pltpu.sync_copy(src.at[pl.ds(base, _CH)], src_v)
            pltpu.sync_copy(dst.at[pl.ds(base, _CH)], dst_v)
            c1 = pltpu.async_copy(qh.at[src_v], qb, sem)
            c2 = pltpu.async_copy(rh.at[dst_v], rb, sem)
            c1.wait()
            c2.wait()

            def row(r, rc):
                for g in range(_D // 16):
                    sl = pl.ds(g * 16, 16)
                    qb[r, sl] = qb[r, sl] + rb[r, sl]
                return rc

            lax.fori_loop(0, _CH, row, 0)
            pltpu.sync_copy(qb, g1.at[pl.ds(base, _CH)])
            return carry

        lax.fori_loop(0, _NCHUNK, body, 0)

    return _gather


@functools.lru_cache(maxsize=1)
def _make_scatter():
    @functools.partial(
        pl.kernel,
        mesh=_sc_mesh(),
        out_type=jax.ShapeDtypeStruct((_NC, _NP, _D), jnp.float32),
        scratch_types=[
            pltpu.VMEM((_CH,), jnp.int32),
            pltpu.VMEM((_CH,), jnp.int32),
            pltpu.VMEM((_CH, _D), jnp.float32),
            pltpu.VMEM((_CH, _D), jnp.float32),
            pltpu.VMEM_SHARED((_NP, _D), jnp.float32),
            pltpu.SemaphoreType.DMA,
        ],
    )
    def _scatter(ehat, vh, src, dst, zeros_hbm, out,
                 src_v, dst_v, eb, vb, agg_sh, sem):
        c = lax.axis_index("c")
        s = lax.axis_index("s")
        wid = s * _NC + c
        pltpu.sync_copy(zeros_hbm.at[pl.ds(s * _RPT, _RPT)],
                        agg_sh.at[pl.ds(s * _RPT, _RPT)])
        plsc.subcore_barrier()
        base0 = wid * _EPW

        def body(i, carry):
            base = base0 + i * _CH
            pltpu.sync_copy(src.at[pl.ds(base, _CH)], src_v)
            pltpu.sync_copy(dst.at[pl.ds(base, _CH)], dst_v)
            cv = pltpu.async_copy(vh.at[dst_v], vb, sem)
            pltpu.sync_copy(ehat.at[pl.ds(base, _CH)], eb)
            cv.wait()

            def row(r, rc):
                for g in range(_D // 16):
                    sl = pl.ds(g * 16, 16)
                    x = eb[r, sl]
                    gate = 1.0 / (1.0 + jnp.exp(-x))
                    vb[r, sl] = gate * vb[r, sl]
                return rc

            lax.fori_loop(0, _CH, row, 0)
            pltpu.sync_copy(vb, agg_sh.at[src_v], add=True)
            return carry

        lax.fori_loop(0, _NCHUNK, body, 0)
        plsc.subcore_barrier()
        pltpu.sync_copy(agg_sh.at[pl.ds(s * _RPT, _RPT)],
                        out.at[c, pl.ds(s * _RPT, _RPT)])

    return _scatter


def kernel(h, e, edge_index, P_w, Q_w, R_w, U_w, V_w,
           m1_w, m1_b, m2_w, m2_b, e_gamma, e_beta, n_gamma, n_beta):
    src = edge_index[0]
    dst = edge_index[1]
    w4 = jnp.concatenate([Q_w.T, R_w.T, V_w.T, U_w.T], axis=1)
    qh, rh, vh, uh = _node_mm(h, w4)
    g1 = _make_gather()(qh, rh, src, dst)
    ehat, stats = _edge_pass1(P_w.T, e, g1)
    aggc = _make_scatter()(ehat, vh, src, dst, jnp.zeros((_NP, _D), jnp.float32))
    e_new = _edge_pass2(stats, m1_w.T, m2_w.T,
                        m1_b.reshape(1, _D), m2_b.reshape(1, _D),
                        e_gamma.reshape(1, _D), e_beta.reshape(1, _D),
                        e, ehat)
    h_new = _node_pass(h, uh, aggc[0, :_N], aggc[1, :_N],
                       n_gamma.reshape(1, _D), n_beta.reshape(1, _D))
    return (h_new, e_new)
